# baseline (speedup 1.0000x reference)
import functools

import jax
import jax.numpy as jnp
from jax.experimental import pallas as pl
from jax.experimental.pallas import tpu as pltpu

_ALPHA = 0.2
_EPS = 1e-5


def _hst_body(x_ref, w_ref, b_ref, asrc_ref, adst_ref, h_ref, s_ref, t_ref):
    h = jax.lax.dot_general(
        x_ref[...], w_ref[...], (((1,), (1,)), ((), ())),
        preferred_element_type=jnp.float32) + b_ref[...]
    h_ref[...] = h
    s_ref[...] = jax.lax.dot_general(
        h, asrc_ref[...], (((1,), (0,)), ((), ())),
        preferred_element_type=jnp.float32)
    t_ref[...] = jax.lax.dot_general(
        h, adst_ref[...], (((1,), (0,)), ((), ())),
        preferred_element_type=jnp.float32)


def _gat_body(adj_ref, sn_ref, sn2_ref, tn_ref, tn2_ref, h_ref, g_ref, be_ref,
              o_ref, acc_ref, *, n, tk, nk):
    k = pl.program_id(1)

    @pl.when(k == 0)
    def _():
        acc_ref[...] = jnp.zeros_like(acc_ref)

    m = jnp.minimum(sn_ref[...] + tn_ref[...], sn2_ref[...] + tn2_ref[...])
    w = adj_ref[...] * jnp.exp(m)

    def _masked():
        tm = w.shape[0]
        col = k * tk + jax.lax.broadcasted_iota(jnp.int32, (tm, tk), 1)
        return jnp.where(col < n, w, 0.0)

    w = jax.lax.cond(k == nk - 1, _masked, lambda: w)
    acc_ref[...] += jax.lax.dot_general(
        w, h_ref[...], (((1,), (0,)), ((), ())),
        preferred_element_type=jnp.float32)

    @pl.when(k == nk - 1)
    def _():
        hp = acc_ref[...]
        mean = jnp.mean(hp, axis=1, keepdims=True)
        c = hp - mean
        var = jnp.mean(c * c, axis=1, keepdims=True)
        hn = c * jax.lax.rsqrt(var + _EPS) * g_ref[...] + be_ref[...]
        o_ref[...] = jnp.where(hn > 0, hn, jnp.exp(jnp.minimum(hn, 0.0)) - 1.0)


def kernel(input, adj, W, b, a, gamma, beta):
    n, f = input.shape

    tm2 = 2000 if n % 2000 == 0 else (128 if n % 128 == 0 else 8)
    asrc = a[0, :f].reshape(f, 1)
    adst = a[0, f:].reshape(f, 1)
    h, s, t = pl.pallas_call(
        _hst_body,
        grid=(n // tm2,),
        in_specs=[
            pl.BlockSpec((tm2, f), lambda i: (i, 0)),
            pl.BlockSpec((f, f), lambda i: (0, 0)),
            pl.BlockSpec((1, f), lambda i: (0, 0)),
            pl.BlockSpec((f, 1), lambda i: (0, 0)),
            pl.BlockSpec((f, 1), lambda i: (0, 0)),
        ],
        out_specs=[
            pl.BlockSpec((tm2, f), lambda i: (i, 0)),
            pl.BlockSpec((tm2, 1), lambda i: (i, 0)),
            pl.BlockSpec((tm2, 1), lambda i: (i, 0)),
        ],
        out_shape=[
            jax.ShapeDtypeStruct((n, f), jnp.float32),
            jax.ShapeDtypeStruct((n, 1), jnp.float32),
            jax.ShapeDtypeStruct((n, 1), jnp.float32),
        ],
    )(input, W, b.reshape(1, f), asrc, adst)

    tm = 1000 if n % 1000 == 0 else (128 if n % 128 == 0 else 8)
    tk = 2048
    nk = pl.cdiv(n, tk)
    npad = nk * tk
    h_pad = jnp.pad(h, ((0, npad - n), (0, 0)))
    sn = -s
    sn2 = -_ALPHA * s
    tpad = jnp.pad(t, ((0, npad - n), (0, 0))).reshape(1, npad)
    tn = -tpad
    tn2 = -_ALPHA * tpad

    out = pl.pallas_call(
        functools.partial(_gat_body, n=n, tk=tk, nk=nk),
        grid=(n // tm, nk),
        in_specs=[
            pl.BlockSpec((tm, tk), lambda i, k: (i, k)),
            pl.BlockSpec((tm, 1), lambda i, k: (i, 0)),
            pl.BlockSpec((tm, 1), lambda i, k: (i, 0)),
            pl.BlockSpec((1, tk), lambda i, k: (0, k)),
            pl.BlockSpec((1, tk), lambda i, k: (0, k)),
            pl.BlockSpec((tk, f), lambda i, k: (k, 0)),
            pl.BlockSpec((1, f), lambda i, k: (0, 0)),
            pl.BlockSpec((1, f), lambda i, k: (0, 0)),
        ],
        out_specs=pl.BlockSpec((tm, f), lambda i, k: (i, 0)),
        out_shape=jax.ShapeDtypeStruct((n, f), jnp.float32),
        scratch_shapes=[pltpu.VMEM((tm, f), jnp.float32)],
    )(adj, sn, sn2, tn, tn2, h_pad, gamma.reshape(1, f), beta.reshape(1, f))
    return out


# speedup vs baseline: 1.6986x; 1.6986x over previous
import functools

import jax
import jax.numpy as jnp
from jax.experimental import pallas as pl
from jax.experimental.pallas import tpu as pltpu

_ALPHA = 0.2
_EPS = 1e-5


def _hst_body(x_ref, w_ref, b_ref, asrc_ref, adst_ref, h_ref, s_ref, t_ref):
    h = jax.lax.dot_general(
        x_ref[...], w_ref[...], (((1,), (1,)), ((), ())),
        preferred_element_type=jnp.float32) + b_ref[...]
    h_ref[...] = h
    s_ref[...] = jax.lax.dot_general(
        h, asrc_ref[...], (((1,), (0,)), ((), ())),
        preferred_element_type=jnp.float32)
    t_ref[...] = jax.lax.dot_general(
        h, adst_ref[...], (((1,), (0,)), ((), ())),
        preferred_element_type=jnp.float32)


def _gat_body(adj_ref, sn_ref, tn_ref, h_ref, g_ref, be_ref, o_ref,
              acc_ref, w_ref, *, n, tm, tk, nk, rc):
    k = pl.program_id(1)

    @pl.when(k == 0)
    def _():
        acc_ref[...] = jnp.zeros_like(acc_ref)

    tn = tn_ref[...]
    for c in range(tm // rc):
        sl = pl.ds(c * rc, rc)
        u = sn_ref[sl, :] + tn
        m = jnp.minimum(u, _ALPHA * u)
        w_ref[sl, :] = adj_ref[sl, :] * jnp.exp2(m)

    tail = n - (nk - 1) * tk
    if tail < tk:
        @pl.when(k == nk - 1)
        def _():
            w_ref[:, pl.ds(tail, tk - tail)] = jnp.zeros(
                (tm, tk - tail), jnp.float32)

    acc_ref[...] += jax.lax.dot_general(
        w_ref[...], h_ref[...], (((1,), (0,)), ((), ())),
        preferred_element_type=jnp.float32)

    @pl.when(k == nk - 1)
    def _():
        rl = 200 if tm % 200 == 0 else rc
        for c in range(tm // rl):
            sl = pl.ds(c * rl, rl)
            hp = acc_ref[sl, :]
            mean = jnp.mean(hp, axis=1, keepdims=True)
            cen = hp - mean
            var = jnp.mean(cen * cen, axis=1, keepdims=True)
            hn = cen * jax.lax.rsqrt(var + _EPS) * g_ref[...] + be_ref[...]
            o_ref[sl, :] = jnp.where(hn > 0, hn, jnp.exp(jnp.minimum(hn, 0.0)) - 1.0)


def kernel(input, adj, W, b, a, gamma, beta):
    n, f = input.shape

    tm2 = 2000 if n % 2000 == 0 else (128 if n % 128 == 0 else 8)
    asrc = a[0, :f].reshape(f, 1)
    adst = a[0, f:].reshape(f, 1)
    h, s, t = pl.pallas_call(
        _hst_body,
        grid=(n // tm2,),
        in_specs=[
            pl.BlockSpec((tm2, f), lambda i: (i, 0)),
            pl.BlockSpec((f, f), lambda i: (0, 0)),
            pl.BlockSpec((1, f), lambda i: (0, 0)),
            pl.BlockSpec((f, 1), lambda i: (0, 0)),
            pl.BlockSpec((f, 1), lambda i: (0, 0)),
        ],
        out_specs=[
            pl.BlockSpec((tm2, f), lambda i: (i, 0)),
            pl.BlockSpec((tm2, 1), lambda i: (i, 0)),
            pl.BlockSpec((tm2, 1), lambda i: (i, 0)),
        ],
        out_shape=[
            jax.ShapeDtypeStruct((n, f), jnp.float32),
            jax.ShapeDtypeStruct((n, 1), jnp.float32),
            jax.ShapeDtypeStruct((n, 1), jnp.float32),
        ],
    )(input, W, b.reshape(1, f), asrc, adst)

    tm = 1000 if n % 1000 == 0 else (128 if n % 128 == 0 else 8)
    rc = 8
    tk = 2048
    nk = pl.cdiv(n, tk)
    npad = nk * tk
    h_pad = jnp.pad(h, ((0, npad - n), (0, 0)))
    log2e = jnp.float32(1.4426950408889634)
    sn = -log2e * s
    tn = (-log2e * jnp.pad(t, ((0, npad - n), (0, 0)))).reshape(1, npad)

    out = pl.pallas_call(
        functools.partial(_gat_body, n=n, tm=tm, tk=tk, nk=nk, rc=rc),
        grid=(n // tm, nk),
        in_specs=[
            pl.BlockSpec((tm, tk), lambda i, k: (i, k)),
            pl.BlockSpec((tm, 1), lambda i, k: (i, 0)),
            pl.BlockSpec((1, tk), lambda i, k: (0, k)),
            pl.BlockSpec((tk, f), lambda i, k: (k, 0)),
            pl.BlockSpec((1, f), lambda i, k: (0, 0)),
            pl.BlockSpec((1, f), lambda i, k: (0, 0)),
        ],
        out_specs=pl.BlockSpec((tm, f), lambda i, k: (i, 0)),
        out_shape=jax.ShapeDtypeStruct((n, f), jnp.float32),
        scratch_shapes=[
            pltpu.VMEM((tm, f), jnp.float32),
            pltpu.VMEM((tm, tk), jnp.float32),
        ],
    )(adj, sn, tn, h_pad, gamma.reshape(1, f), beta.reshape(1, f))
    return out


# speedup vs baseline: 1.8856x; 1.1101x over previous
import functools

import jax
import jax.numpy as jnp
from jax.experimental import pallas as pl
from jax.experimental.pallas import tpu as pltpu

_ALPHA = 0.2
_EPS = 1e-5


def _hst_body(x_ref, w_ref, b_ref, asrc_ref, adst_ref,
              h_ref, p_ref, pa_ref, q_ref, qa_ref):
    h = jax.lax.dot_general(
        x_ref[...], w_ref[...], (((1,), (1,)), ((), ())),
        preferred_element_type=jnp.float32) + b_ref[...]
    h_ref[...] = h
    s = jax.lax.dot_general(
        h, asrc_ref[...], (((1,), (0,)), ((), ())),
        preferred_element_type=jnp.float32)
    t = jax.lax.dot_general(
        h, adst_ref[...], (((1,), (0,)), ((), ())),
        preferred_element_type=jnp.float32)
    p_ref[...] = jnp.exp(-s)
    pa_ref[...] = jnp.exp(-_ALPHA * s)
    q_ref[...] = jnp.exp(-t)
    qa_ref[...] = jnp.exp(-_ALPHA * t)


def _gat_body(adj_ref, p_ref, pa_ref, q_ref, qa_ref, h_ref, g_ref, be_ref,
              o_ref, acc_ref, w_ref, *, n, tm, tk, nk, rc):
    k = pl.program_id(1)

    @pl.when(k == 0)
    def _():
        acc_ref[...] = jnp.zeros_like(acc_ref)

    q = q_ref[...]
    qa = qa_ref[...]
    for c in range(tm // rc):
        sl = pl.ds(c * rc, rc)
        m1 = p_ref[sl, :] * q
        m2 = pa_ref[sl, :] * qa
        w_ref[sl, :] = adj_ref[sl, :] * jnp.minimum(m1, m2)

    tail = n - (nk - 1) * tk
    if tail < tk:
        @pl.when(k == nk - 1)
        def _():
            w_ref[:, pl.ds(tail, tk - tail)] = jnp.zeros(
                (tm, tk - tail), jnp.float32)

    acc_ref[...] += jax.lax.dot_general(
        w_ref[...], h_ref[...], (((1,), (0,)), ((), ())),
        preferred_element_type=jnp.float32)

    @pl.when(k == nk - 1)
    def _():
        rl = 200 if tm % 200 == 0 else rc
        for c in range(tm // rl):
            sl = pl.ds(c * rl, rl)
            hp = acc_ref[sl, :]
            mean = jnp.mean(hp, axis=1, keepdims=True)
            cen = hp - mean
            var = jnp.mean(cen * cen, axis=1, keepdims=True)
            hn = cen * jax.lax.rsqrt(var + _EPS) * g_ref[...] + be_ref[...]
            o_ref[sl, :] = jnp.where(hn > 0, hn, jnp.exp(jnp.minimum(hn, 0.0)) - 1.0)


def kernel(input, adj, W, b, a, gamma, beta):
    n, f = input.shape

    tm2 = 2000 if n % 2000 == 0 else (128 if n % 128 == 0 else 8)
    asrc = a[0, :f].reshape(f, 1)
    adst = a[0, f:].reshape(f, 1)
    col = jax.ShapeDtypeStruct((n, 1), jnp.float32)
    h, p, pa, q, qa = pl.pallas_call(
        _hst_body,
        grid=(n // tm2,),
        in_specs=[
            pl.BlockSpec((tm2, f), lambda i: (i, 0)),
            pl.BlockSpec((f, f), lambda i: (0, 0)),
            pl.BlockSpec((1, f), lambda i: (0, 0)),
            pl.BlockSpec((f, 1), lambda i: (0, 0)),
            pl.BlockSpec((f, 1), lambda i: (0, 0)),
        ],
        out_specs=[
            pl.BlockSpec((tm2, f), lambda i: (i, 0)),
            pl.BlockSpec((tm2, 1), lambda i: (i, 0)),
            pl.BlockSpec((tm2, 1), lambda i: (i, 0)),
            pl.BlockSpec((tm2, 1), lambda i: (i, 0)),
            pl.BlockSpec((tm2, 1), lambda i: (i, 0)),
        ],
        out_shape=[jax.ShapeDtypeStruct((n, f), jnp.float32),
                   col, col, col, col],
        compiler_params=pltpu.CompilerParams(
            dimension_semantics=("parallel",)),
    )(input, W, b.reshape(1, f), asrc, adst)

    tm = 2000 if n % 2000 == 0 else (128 if n % 128 == 0 else 8)
    rc = 8
    tk = 2048
    nk = pl.cdiv(n, tk)
    npad = nk * tk
    h_pad = jnp.pad(h, ((0, npad - n), (0, 0)))
    q_row = jnp.pad(q, ((0, npad - n), (0, 0))).reshape(1, npad)
    qa_row = jnp.pad(qa, ((0, npad - n), (0, 0))).reshape(1, npad)

    out = pl.pallas_call(
        functools.partial(_gat_body, n=n, tm=tm, tk=tk, nk=nk, rc=rc),
        grid=(n // tm, nk),
        in_specs=[
            pl.BlockSpec((tm, tk), lambda i, k: (i, k)),
            pl.BlockSpec((tm, 1), lambda i, k: (i, 0)),
            pl.BlockSpec((tm, 1), lambda i, k: (i, 0)),
            pl.BlockSpec((1, tk), lambda i, k: (0, k)),
            pl.BlockSpec((1, tk), lambda i, k: (0, k)),
            pl.BlockSpec((tk, f), lambda i, k: (k, 0)),
            pl.BlockSpec((1, f), lambda i, k: (0, 0)),
            pl.BlockSpec((1, f), lambda i, k: (0, 0)),
        ],
        out_specs=pl.BlockSpec((tm, f), lambda i, k: (i, 0)),
        out_shape=jax.ShapeDtypeStruct((n, f), jnp.float32),
        scratch_shapes=[
            pltpu.VMEM((tm, f), jnp.float32),
            pltpu.VMEM((tm, tk), jnp.float32),
        ],
        compiler_params=pltpu.CompilerParams(
            dimension_semantics=("parallel", "arbitrary"),
            vmem_limit_bytes=110 * 1024 * 1024),
    )(adj, p, pa, q_row, qa_row, h_pad,
      gamma.reshape(1, f), beta.reshape(1, f))
    return out


# speedup vs baseline: 2.2165x; 1.1755x over previous
import functools

import jax
import jax.numpy as jnp
from jax.experimental import pallas as pl
from jax.experimental.pallas import tpu as pltpu

_ALPHA = 0.2
_EPS = 1e-5


def _hst_body(x_ref, w_ref, b_ref, asrc_ref, adst_ref,
              h_ref, p_ref, pa_ref, q_ref, qa_ref, *, n, tm2):
    i = pl.program_id(0)
    h = jax.lax.dot_general(
        x_ref[...], w_ref[...], (((1,), (1,)), ((), ())),
        preferred_element_type=jnp.float32) + b_ref[...]
    row = i * tm2 + jax.lax.broadcasted_iota(jnp.int32, (tm2, 1), 0)
    h = jnp.where(row < n, h, 0.0)
    h_ref[...] = h.astype(jnp.bfloat16)
    s = jax.lax.dot_general(
        h, asrc_ref[...], (((1,), (0,)), ((), ())),
        preferred_element_type=jnp.float32)
    t = jax.lax.dot_general(
        h, adst_ref[...], (((1,), (0,)), ((), ())),
        preferred_element_type=jnp.float32)
    p_ref[...] = jnp.exp(-s)
    pa_ref[...] = jnp.exp(-_ALPHA * s)
    q_ref[...] = jnp.exp(-t)
    qa_ref[...] = jnp.exp(-_ALPHA * t)


def _gat_body(adj_ref, p_ref, pa_ref, q_ref, qa_ref, h_ref, g_ref, be_ref,
              o_ref, wa_ref, wb_ref, *, n, tm, tk, nk, rc):
    tail = n - (nk - 1) * tk
    acc = jnp.zeros((tm, h_ref.shape[1]), jnp.float32)
    for k in range(nk):
        w_ref = wa_ref if k % 2 == 0 else wb_ref
        q = q_ref[:, pl.ds(k * tk, tk)]
        qa = qa_ref[:, pl.ds(k * tk, tk)]
        for c in range(tm // rc):
            sl = pl.ds(c * rc, rc)
            m1 = p_ref[sl, :] * q
            m2 = pa_ref[sl, :] * qa
            w = adj_ref[sl, pl.ds(k * tk, tk)] * jnp.minimum(m1, m2)
            w_ref[sl, :] = w.astype(jnp.bfloat16)
        if k == nk - 1 and tail < tk:
            w_ref[:, pl.ds(tail, tk - tail)] = jnp.zeros(
                (tm, tk - tail), jnp.bfloat16)
        acc = acc + jax.lax.dot_general(
            w_ref[...], h_ref[pl.ds(k * tk, tk), :], (((1,), (0,)), ((), ())),
            preferred_element_type=jnp.float32)

    mean = jnp.mean(acc, axis=1, keepdims=True)
    cen = acc - mean
    var = jnp.mean(cen * cen, axis=1, keepdims=True)
    hn = cen * jax.lax.rsqrt(var + _EPS) * g_ref[...] + be_ref[...]
    o_ref[...] = jnp.where(hn > 0, hn, jnp.exp(jnp.minimum(hn, 0.0)) - 1.0)


def kernel(input, adj, W, b, a, gamma, beta):
    n, f = input.shape
    tk = 2048
    nk = pl.cdiv(n, tk)
    npad = nk * tk

    tm2 = npad // 5 if npad % 5 == 0 else npad
    asrc = a[0, :f].reshape(f, 1)
    adst = a[0, f:].reshape(f, 1)
    col = jax.ShapeDtypeStruct((npad, 1), jnp.float32)
    h, p, pa, q, qa = pl.pallas_call(
        functools.partial(_hst_body, n=n, tm2=tm2),
        grid=(npad // tm2,),
        in_specs=[
            pl.BlockSpec((tm2, f), lambda i: (i, 0)),
            pl.BlockSpec((f, f), lambda i: (0, 0)),
            pl.BlockSpec((1, f), lambda i: (0, 0)),
            pl.BlockSpec((f, 1), lambda i: (0, 0)),
            pl.BlockSpec((f, 1), lambda i: (0, 0)),
        ],
        out_specs=[
            pl.BlockSpec((tm2, f), lambda i: (i, 0)),
            pl.BlockSpec((tm2, 1), lambda i: (i, 0)),
            pl.BlockSpec((tm2, 1), lambda i: (i, 0)),
            pl.BlockSpec((tm2, 1), lambda i: (i, 0)),
            pl.BlockSpec((tm2, 1), lambda i: (i, 0)),
        ],
        out_shape=[jax.ShapeDtypeStruct((npad, f), jnp.bfloat16),
                   col, col, col, col],
        compiler_params=pltpu.CompilerParams(
            dimension_semantics=("parallel",)),
    )(input, W, b.reshape(1, f), asrc, adst)

    q_row = q.reshape(1, npad)
    qa_row = qa.reshape(1, npad)

    tm = 400 if n % 400 == 0 else (128 if n % 128 == 0 else 8)
    rc = 16 if tm % 16 == 0 else 8

    out = pl.pallas_call(
        functools.partial(_gat_body, n=n, tm=tm, tk=tk, nk=nk, rc=rc),
        grid=(n // tm,),
        in_specs=[
            pl.BlockSpec((tm, npad), lambda i: (i, 0)),
            pl.BlockSpec((tm, 1), lambda i: (i, 0)),
            pl.BlockSpec((tm, 1), lambda i: (i, 0)),
            pl.BlockSpec((1, npad), lambda i: (0, 0)),
            pl.BlockSpec((1, npad), lambda i: (0, 0)),
            pl.BlockSpec((npad, f), lambda i: (0, 0)),
            pl.BlockSpec((1, f), lambda i: (0, 0)),
            pl.BlockSpec((1, f), lambda i: (0, 0)),
        ],
        out_specs=pl.BlockSpec((tm, f), lambda i: (i, 0)),
        out_shape=jax.ShapeDtypeStruct((n, f), jnp.float32),
        scratch_shapes=[
            pltpu.VMEM((tm, tk), jnp.bfloat16),
            pltpu.VMEM((tm, tk), jnp.bfloat16),
        ],
        compiler_params=pltpu.CompilerParams(
            dimension_semantics=("parallel",),
            vmem_limit_bytes=110 * 1024 * 1024),
    )(adj, p, pa, q_row, qa_row, h,
      gamma.reshape(1, f), beta.reshape(1, f))
    return out
